# baseline (speedup 1.0000x reference)
import functools

import jax
import jax.numpy as jnp
from jax import lax
from jax.experimental import pallas as pl
from jax.experimental.pallas import tpu as pltpu
from jax.experimental.pallas import tpu_sc as plsc

F = 26
V = 100000
D = 32
B = 16384
L = 16

NC, NS = 2, 16
NW = NC * NS
BPW = B // NW

CB = 32
NCHUNK = BPW // CB
IPC = CB * F
NBG = CB // L

_GS = [(s, min(128, IPC - s)) for s in range(0, IPC, 128)]

_mesh = plsc.VectorSubcoreMesh(core_axis_name="c", subcore_axis_name="s")


@functools.partial(
    pl.kernel,
    mesh=_mesh,
    out_type=jax.ShapeDtypeStruct((B * D // 128, 128), jnp.float32),
    scratch_types=[
        pltpu.VMEM((IPC,), jnp.int32),
        pltpu.VMEM((IPC,), jnp.int32),
        pltpu.VMEM((IPC,), jnp.int32),
        pltpu.VMEM((IPC, 128), jnp.float32),
        pltpu.VMEM((CB * D // 128, 128), jnp.float32),
        pltpu.SemaphoreType.DMA,
    ],
    compiler_params=pltpu.CompilerParams(
        use_tc_tiling_on_sc=True, needs_layout_passes=False
    ),
)
def _emb_kernel(c_hbm, t_hbm, out_hbm, craw_v, line_v, offs_v, rows_v,
                acc_v, sem):
    wid = lax.axis_index("s") * NC + lax.axis_index("c")

    for v in range(IPC // L):
        p = v * L + lax.iota(jnp.int32, L)
        offs_v[pl.ds(v * L, L)] = (p % F) * V

    def chunk_body(ci, carry):
        cb0 = wid * BPW + ci * CB
        i0 = wid * (BPW * F) + ci * IPC

        pltpu.sync_copy(c_hbm.at[pl.ds(i0, IPC)], craw_v)
        for v in range(IPC // L):
            sl = pl.ds(v * L, L)
            line_v[sl] = (craw_v[sl] + offs_v[sl]) >> 2

        copies = [
            pltpu.async_copy(
                t_hbm.at[line_v.at[pl.ds(s, n)]],
                rows_v.at[pl.ds(s, n), :],
                sem,
            )
            for (s, n) in _GS
        ]
        for cp in copies:
            cp.wait()

        for bg in range(NBG):
            lanes = lax.iota(jnp.int32, L)

            def f_body(f, accs):
                cv = plsc.load_gather(
                    craw_v, [bg * (L * F) + lanes * F + f])
                row = bg * (L * F) + lanes * F + f
                colbase = (cv & 3) * D
                new = []
                for d in range(D):
                    col = colbase + d
                    g = plsc.load_gather(rows_v, [row, col])
                    new.append(accs[d] + g)
                return tuple(new)

            accs = lax.fori_loop(
                0, F, f_body,
                tuple(jnp.zeros((L,), jnp.float32) for _ in range(D)))

            b_local = bg * L + lanes
            for d in range(D):
                pos = b_local * D + d
                plsc.store_scatter(
                    acc_v, [pos >> 7, pos & 127],
                    accs[d] * jnp.float32(1.0 / F))

        out_r0 = pl.multiple_of(cb0 * D // 128, 8)
        pltpu.sync_copy(acc_v, out_hbm.at[pl.ds(out_r0, CB * D // 128), :])
        return carry

    lax.fori_loop(0, NCHUNK, chunk_body, 0)


def kernel(c, tables):
    lines = tables.reshape(F * V // 4, 128)
    c_flat = c.reshape(B * F)
    out = _emb_kernel(c_flat, lines)
    return out.reshape(B, D)


# speedup vs baseline: 3.0581x; 3.0581x over previous
import functools

import jax
import jax.numpy as jnp
from jax import lax
from jax.experimental import pallas as pl
from jax.experimental.pallas import tpu as pltpu
from jax.experimental.pallas import tpu_sc as plsc

F = 26
V = 100000
D = 32
B = 16384
L = 16

NC, NS = 2, 16
NW = NC * NS

CQ = 4096
NQ = B // CQ

_mesh = plsc.VectorSubcoreMesh(core_axis_name="c", subcore_axis_name="s")


@functools.partial(
    pl.kernel,
    mesh=_mesh,
    out_type=jax.ShapeDtypeStruct((D, B), jnp.float32),
    scratch_types=[
        pltpu.VMEM((V,), jnp.float32),
        pltpu.VMEM((CQ,), jnp.int32),
        pltpu.VMEM((B,), jnp.float32),
        pltpu.SemaphoreType.DMA,
    ],
    compiler_params=pltpu.CompilerParams(needs_layout_passes=False),
)
def _emb_kernel(ct_hbm, tt_hbm, out_hbm, row_v, cq_v, acc_v, sem):
    d = lax.axis_index("s") * NC + lax.axis_index("c")

    def f_body(f, carry):
        pltpu.sync_copy(tt_hbm.at[f, d, :], row_v)
        first = f == 0

        def q_body(q, carry2):
            pltpu.sync_copy(ct_hbm.at[f, pl.ds(q * CQ, CQ)], cq_v)

            def k_body(k, carry3):
                idx = cq_v[pl.ds(k * L, L)]
                g = plsc.load_gather(row_v, [idx])
                sl = pl.ds(q * CQ + k * L, L)

                @pl.when(first)
                def _():
                    acc_v[sl] = g

                @pl.when(jnp.logical_not(first))
                def _():
                    plsc.addupdate(acc_v.at[sl], g)

                return carry3

            lax.fori_loop(0, CQ // L, k_body, 0)
            return carry2

        lax.fori_loop(0, NQ, q_body, 0)
        return carry

    lax.fori_loop(0, F, f_body, 0)

    def s_body(k, carry):
        sl = pl.ds(k * L, L)
        acc_v[sl] = acc_v[sl] * jnp.float32(1.0 / F)
        return carry

    lax.fori_loop(0, B // L, s_body, 0)
    pltpu.sync_copy(acc_v, out_hbm.at[d, :])


def kernel(c, tables):
    tt = tables.transpose(0, 2, 1)
    ct = c.T
    out_t = _emb_kernel(ct, tt)
    return out_t.T


# speedup vs baseline: 3.1919x; 1.0438x over previous
import functools

import jax
import jax.numpy as jnp
from jax import lax
from jax.experimental import pallas as pl
from jax.experimental.pallas import tpu as pltpu
from jax.experimental.pallas import tpu_sc as plsc

F = 26
V = 100000
D = 32
B = 16384
L = 16

NC, NS = 2, 16
NW = NC * NS

CQ = 4096
NQ = B // CQ

_mesh = plsc.VectorSubcoreMesh(core_axis_name="c", subcore_axis_name="s")


@functools.partial(
    pl.kernel,
    mesh=_mesh,
    out_type=jax.ShapeDtypeStruct((D, B), jnp.float32),
    scratch_types=[
        pltpu.VMEM((V,), jnp.float32),
        pltpu.VMEM((CQ,), jnp.int32),
        pltpu.VMEM((B,), jnp.float32),
        pltpu.SemaphoreType.DMA,
    ],
    compiler_params=pltpu.CompilerParams(needs_layout_passes=False),
)
def _emb_kernel(ct_hbm, tt_hbm, out_hbm, row_v, cq_v, acc_v, sem):
    d = lax.axis_index("s") * NC + lax.axis_index("c")

    def make_q_body(f, store):
        def q_body(q, carry2):
            pltpu.sync_copy(ct_hbm.at[f, pl.ds(q * CQ, CQ)], cq_v)

            def k_body(k, carry3):
                idx = cq_v[pl.ds(k * L, L)]
                g = plsc.load_gather(row_v, [idx])
                sl = pl.ds(q * CQ + k * L, L)
                if store:
                    acc_v[sl] = g
                else:
                    plsc.addupdate(acc_v.at[sl], g)
                return carry3

            lax.fori_loop(0, CQ // L, k_body, 0, unroll=8)
            return carry2

        return q_body

    pltpu.sync_copy(tt_hbm.at[0, d, :], row_v)
    lax.fori_loop(0, NQ, make_q_body(0, True), 0)

    def f_body(f, carry):
        pltpu.sync_copy(tt_hbm.at[f, d, :], row_v)
        lax.fori_loop(0, NQ, make_q_body(f, False), 0)
        return carry

    lax.fori_loop(1, F, f_body, 0)

    def s_body(k, carry):
        sl = pl.ds(k * L, L)
        acc_v[sl] = acc_v[sl] * jnp.float32(1.0 / F)
        return carry

    lax.fori_loop(0, B // L, s_body, 0)
    pltpu.sync_copy(acc_v, out_hbm.at[d, :])


def kernel(c, tables):
    tt = tables.transpose(0, 2, 1)
    ct = c.T
    out_t = _emb_kernel(ct, tt)
    return out_t.T


# speedup vs baseline: 3.4293x; 1.0744x over previous
import functools

import jax
import jax.numpy as jnp
from jax import lax
from jax.experimental import pallas as pl
from jax.experimental.pallas import tpu as pltpu
from jax.experimental.pallas import tpu_sc as plsc

F = 26
V = 100000
D = 32
B = 16384
L = 16

NC, NS = 2, 16
NW = NC * NS

CQ = 8192
NQ = B // CQ

_mesh = plsc.VectorSubcoreMesh(core_axis_name="c", subcore_axis_name="s")


@functools.partial(
    pl.kernel,
    mesh=_mesh,
    out_type=jax.ShapeDtypeStruct((D, B), jnp.float32),
    scratch_types=[
        pltpu.VMEM((V,), jnp.float32),
        pltpu.VMEM((CQ,), jnp.int32),
        pltpu.VMEM((B,), jnp.float32),
        pltpu.SemaphoreType.DMA,
    ],
    compiler_params=pltpu.CompilerParams(needs_layout_passes=False),
)
def _emb_kernel(ct_hbm, tt_hbm, out_hbm, row_v, cq_v, acc_v, sem):
    d = lax.axis_index("s") * NC + lax.axis_index("c")

    def make_q_body(f, store):
        def q_body(q, carry2):
            pltpu.sync_copy(ct_hbm.at[f, pl.ds(q * CQ, CQ)], cq_v)

            def k_body(k, carry3):
                idx = cq_v[pl.ds(k * L, L)]
                g = plsc.load_gather(row_v, [idx])
                sl = pl.ds(q * CQ + k * L, L)
                if store:
                    acc_v[sl] = g
                else:
                    plsc.addupdate(acc_v.at[sl], g)
                return carry3

            lax.fori_loop(0, CQ // L, k_body, 0, unroll=16)
            return carry2

        return q_body

    pltpu.sync_copy(tt_hbm.at[0, d, :], row_v)
    lax.fori_loop(0, NQ, make_q_body(0, True), 0)

    def f_body(f, carry):
        pltpu.sync_copy(tt_hbm.at[f, d, :], row_v)
        lax.fori_loop(0, NQ, make_q_body(f, False), 0)
        return carry

    lax.fori_loop(1, F, f_body, 0)

    def s_body(k, carry):
        sl = pl.ds(k * L, L)
        acc_v[sl] = acc_v[sl] * jnp.float32(1.0 / F)
        return carry

    lax.fori_loop(0, B // L, s_body, 0)
    pltpu.sync_copy(acc_v, out_hbm.at[d, :])


def kernel(c, tables):
    tt = tables.transpose(0, 2, 1)
    ct = c.T
    out_t = _emb_kernel(ct, tt)
    return out_t.T


# speedup vs baseline: 5.3361x; 1.5560x over previous
import functools

import jax
import jax.numpy as jnp
from jax import lax
from jax.experimental import pallas as pl
from jax.experimental.pallas import tpu as pltpu
from jax.experimental.pallas import tpu_sc as plsc

F = 26
V = 100000
D = 32
B = 16384
L = 16

NC, NS = 2, 16
NW = NC * NS

CQ = 4096
NQ = B // CQ

_mesh = plsc.VectorSubcoreMesh(core_axis_name="c", subcore_axis_name="s")


@functools.partial(
    pl.kernel,
    mesh=_mesh,
    out_type=jax.ShapeDtypeStruct((D, B), jnp.float32),
    scratch_types=[
        pltpu.VMEM((V,), jnp.float32),
        pltpu.VMEM((CQ,), jnp.int32),
        pltpu.VMEM((CQ,), jnp.int32),
        pltpu.VMEM((B,), jnp.float32),
        pltpu.SemaphoreType.DMA,
        pltpu.SemaphoreType.DMA,
        pltpu.SemaphoreType.DMA,
    ],
    compiler_params=pltpu.CompilerParams(needs_layout_passes=False),
)
def _emb_kernel(ct_hbm, tt_hbm, out_hbm, row_v, cq0_v, cq1_v, acc_v,
                sem_r, sem_c0, sem_c1):
    d = lax.axis_index("s") * NC + lax.axis_index("c")
    bufs = (cq0_v, cq1_v)
    sems = (sem_c0, sem_c1)

    def field(f, store):
        row_cp = pltpu.async_copy(tt_hbm.at[f, d, :], row_v, sem_r)
        cps = [
            pltpu.async_copy(ct_hbm.at[f, pl.ds(0, CQ)], bufs[0], sems[0]),
            None,
        ]
        row_cp.wait()
        for q in range(NQ):
            b = q % 2
            cps[b].wait()
            if q + 1 < NQ:
                nb = (q + 1) % 2
                cps[nb] = pltpu.async_copy(
                    ct_hbm.at[f, pl.ds((q + 1) * CQ, CQ)], bufs[nb],
                    sems[nb])
            cq_v = bufs[b]

            def k_body(k, carry3):
                idx = cq_v[pl.ds(k * L, L)]
                g = plsc.load_gather(row_v, [idx])
                sl = pl.ds(q * CQ + k * L, L)
                if store:
                    acc_v[sl] = g
                else:
                    plsc.addupdate(acc_v.at[sl], g)
                return carry3

            lax.fori_loop(0, CQ // L, k_body, 0, unroll=16)

    field(0, True)

    def f_body(f, carry):
        field(f, False)
        return carry

    lax.fori_loop(1, F, f_body, 0)

    def s_body(k, carry):
        sl = pl.ds(k * L, L)
        acc_v[sl] = acc_v[sl] * jnp.float32(1.0 / F)
        return carry

    lax.fori_loop(0, B // L, s_body, 0, unroll=8)
    pltpu.sync_copy(acc_v, out_hbm.at[d, :])


def kernel(c, tables):
    tt = tables.transpose(0, 2, 1)
    ct = c.T
    out_t = _emb_kernel(ct, tt)
    return out_t.T
